# baseline (speedup 1.0000x reference)
import functools

import jax
import jax.numpy as jnp
from jax.experimental import pallas as pl
from jax.experimental.pallas import tpu as pltpu
from jax.experimental.pallas import tpu_sc as plsc

_BN = 256

_PREC = jax.lax.Precision.DEFAULT


def _assign_body(x_ref, cbt_ref, enc_ref, b2_ref):
    i = pl.program_id(0)
    k = cbt_ref.shape[1]

    @pl.when(i == 0)
    def _():
        cbt = cbt_ref[...]
        b2_ref[...] = jnp.sum(cbt * cbt, axis=0, keepdims=True)

    x = x_ref[...]
    a2 = jnp.sum(x * x, axis=1, keepdims=True)
    s = jax.lax.dot_general(
        x, cbt_ref[...], (((1,), (0,)), ((), ())),
        preferred_element_type=jnp.float32, precision=_PREC,
    )
    d2 = (a2 - 2.0 * s) + b2_ref[...]
    m = jnp.min(d2, axis=1, keepdims=True)
    thr = jnp.where(m > 0, m * (1.0 + 0.9e-7), 0.0)
    iota = jax.lax.broadcasted_iota(jnp.int32, d2.shape, 1)
    enc_ref[...] = jnp.min(jnp.where(d2 <= thr, iota, k), axis=1, keepdims=True)


@functools.partial(jax.jit, static_argnums=())
def _assign(x, cbt):
    n, d = x.shape
    k = cbt.shape[1]
    return pl.pallas_call(
        _assign_body,
        grid=(n // _BN,),
        in_specs=[
            pl.BlockSpec((_BN, d), lambda i: (i, 0)),
            pl.BlockSpec((d, k), lambda i: (0, 0)),
        ],
        out_specs=pl.BlockSpec((_BN, 1), lambda i: (i, 0)),
        out_shape=jax.ShapeDtypeStruct((n, 1), jnp.int32),
        scratch_shapes=[pltpu.VMEM((1, k), jnp.float32)],
    )(x, cbt)


_GATHER_W = 128


def _sc_gather(codebook, idx2d):
    n = idx2d.shape[1]
    d = codebook.shape[1]
    mesh = plsc.VectorSubcoreMesh(core_axis_name="c", subcore_axis_name="s")

    @functools.partial(
        pl.kernel,
        out_type=jax.ShapeDtypeStruct((n, d), codebook.dtype),
        mesh=mesh,
    )
    def _gather_kernel(cb_hbm, i_hbm, o_hbm):
        def body(i_vmem, o_vmem):
            pltpu.sync_copy(cb_hbm.at[i_vmem.at[0]], o_vmem)

        pltpu.emit_pipeline(
            body,
            grid=(n // _GATHER_W,),
            in_specs=[pl.BlockSpec((1, _GATHER_W), lambda i: (0, i))],
            out_specs=[pl.BlockSpec((_GATHER_W, d), lambda i: (i, 0))],
            core_axis_name=("c", "s"),
            dimension_semantics=(pltpu.PARALLEL,),
        )(i_hbm, o_hbm)

    return _gather_kernel(codebook, idx2d)


_CHUNKS = 4


def kernel(x, codebook):
    n = x.shape[0]
    cbt = codebook.T
    encs, qs = [], []
    for xc in jnp.split(x, _CHUNKS):
        enc = _assign(xc, cbt)
        encs.append(enc)
        qs.append(_sc_gather(codebook, enc.reshape(1, -1)))
    encoded = jnp.concatenate(encs).reshape(n)
    quantized = jnp.concatenate(qs)
    return (encoded, quantized)


# speedup vs baseline: 1.0034x; 1.0034x over previous
import functools

import jax
import jax.numpy as jnp
from jax.experimental import pallas as pl
from jax.experimental.pallas import tpu as pltpu
from jax.experimental.pallas import tpu_sc as plsc

_BN = 256

_PREC = jax.lax.Precision.DEFAULT


def _assign_body(x_ref, cbt_ref, enc_ref, b2_ref):
    i = pl.program_id(0)
    k = cbt_ref.shape[1]

    @pl.when(i == 0)
    def _():
        cbt = cbt_ref[...]
        b2_ref[...] = jnp.sum(cbt * cbt, axis=0, keepdims=True)

    x = x_ref[...]
    a2 = jnp.sum(x * x, axis=1, keepdims=True)
    s = jax.lax.dot_general(
        x, cbt_ref[...], (((1,), (0,)), ((), ())),
        preferred_element_type=jnp.float32, precision=_PREC,
    )
    d2 = (a2 - 2.0 * s) + b2_ref[...]
    m = jnp.min(d2, axis=1, keepdims=True)
    thr = jnp.where(m > 0, m * (1.0 + 0.9e-7), 0.0)
    iota = jax.lax.broadcasted_iota(jnp.int32, d2.shape, 1)
    idx = jnp.min(jnp.where(d2 <= thr, iota, k), axis=1, keepdims=True)
    enc_ref[...] = idx.reshape(1, idx.shape[0])


@functools.partial(jax.jit, static_argnums=())
def _assign(x, cbt):
    n, d = x.shape
    k = cbt.shape[1]
    return pl.pallas_call(
        _assign_body,
        grid=(n // _BN,),
        in_specs=[
            pl.BlockSpec((_BN, d), lambda i: (i, 0)),
            pl.BlockSpec((d, k), lambda i: (0, 0)),
        ],
        out_specs=pl.BlockSpec((1, _BN), lambda i: (0, i)),
        out_shape=jax.ShapeDtypeStruct((1, n), jnp.int32),
        scratch_shapes=[pltpu.VMEM((1, k), jnp.float32)],
    )(x, cbt)


_GATHER_W = 128


def _sc_gather(codebook, idx2d):
    n = idx2d.shape[1]
    d = codebook.shape[1]
    mesh = plsc.VectorSubcoreMesh(core_axis_name="c", subcore_axis_name="s")

    @functools.partial(
        pl.kernel,
        out_type=jax.ShapeDtypeStruct((n, d), codebook.dtype),
        mesh=mesh,
    )
    def _gather_kernel(cb_hbm, i_hbm, o_hbm):
        def body(i_vmem, o_vmem):
            pltpu.sync_copy(cb_hbm.at[i_vmem.at[0]], o_vmem)

        pltpu.emit_pipeline(
            body,
            grid=(n // _GATHER_W,),
            in_specs=[pl.BlockSpec((1, _GATHER_W), lambda i: (0, i))],
            out_specs=[pl.BlockSpec((_GATHER_W, d), lambda i: (i, 0))],
            core_axis_name=("c", "s"),
            dimension_semantics=(pltpu.PARALLEL,),
        )(i_hbm, o_hbm)

    return _gather_kernel(codebook, idx2d)


_CHUNKS = 4


def kernel(x, codebook):
    n = x.shape[0]
    cbt = codebook.T
    encs = [_assign(xc, cbt) for xc in jnp.split(x, _CHUNKS)]
    qs = [_sc_gather(codebook, enc) for enc in encs]
    encoded = jnp.concatenate(encs, axis=1).reshape(n)
    quantized = jnp.concatenate(qs)
    return (encoded, quantized)


# speedup vs baseline: 1.0113x; 1.0079x over previous
import functools

import jax
import jax.numpy as jnp
from jax.experimental import pallas as pl
from jax.experimental.pallas import tpu as pltpu
from jax.experimental.pallas import tpu_sc as plsc

_BN = 256

_PREC = jax.lax.Precision.DEFAULT


def _b2_body(cbt_ref, b2_ref):
    cbt = cbt_ref[...]
    b2_ref[...] = jnp.sum(cbt * cbt, axis=0, keepdims=True)


def _codeword_norms(cbt):
    d, k = cbt.shape
    return pl.pallas_call(
        _b2_body,
        out_shape=jax.ShapeDtypeStruct((1, k), jnp.float32),
    )(cbt)


def _assign_body(x_ref, cbt_ref, b2_ref, enc_ref):
    k = cbt_ref.shape[1]

    x = x_ref[...]
    a2 = jnp.sum(x * x, axis=1, keepdims=True)
    s = jax.lax.dot_general(
        x.astype(jnp.bfloat16), cbt_ref[...], (((1,), (0,)), ((), ())),
        preferred_element_type=jnp.float32, precision=_PREC,
    )
    d2 = (a2 - 2.0 * s) + b2_ref[...]
    m = jnp.min(d2, axis=1, keepdims=True)
    thr = jnp.where(m > 0, m * (1.0 + 0.9e-7), 0.0)
    iota = jax.lax.broadcasted_iota(jnp.int32, d2.shape, 1)
    idx = jnp.min(jnp.where(d2 <= thr, iota, k), axis=1, keepdims=True)
    enc_ref[...] = idx.reshape(1, idx.shape[0])


def _assign(x, cbt_bf, b2):
    n, d = x.shape
    k = cbt_bf.shape[1]
    return pl.pallas_call(
        _assign_body,
        grid=(n // _BN,),
        in_specs=[
            pl.BlockSpec((_BN, d), lambda i: (i, 0)),
            pl.BlockSpec((d, k), lambda i: (0, 0)),
            pl.BlockSpec((1, k), lambda i: (0, 0)),
        ],
        out_specs=pl.BlockSpec((1, _BN), lambda i: (0, i)),
        out_shape=jax.ShapeDtypeStruct((1, n), jnp.int32),
    )(x, cbt_bf, b2)


_GATHER_W = 128


def _sc_gather(codebook, idx2d):
    n = idx2d.shape[1]
    d = codebook.shape[1]
    mesh = plsc.VectorSubcoreMesh(core_axis_name="c", subcore_axis_name="s")

    @functools.partial(
        pl.kernel,
        out_type=jax.ShapeDtypeStruct((n, d), codebook.dtype),
        mesh=mesh,
    )
    def _gather_kernel(cb_hbm, i_hbm, o_hbm):
        def body(i_vmem, o_vmem):
            pltpu.sync_copy(cb_hbm.at[i_vmem.at[0]], o_vmem)

        pltpu.emit_pipeline(
            body,
            grid=(n // _GATHER_W,),
            in_specs=[pl.BlockSpec((1, _GATHER_W), lambda i: (0, i))],
            out_specs=[pl.BlockSpec((_GATHER_W, d), lambda i: (i, 0))],
            core_axis_name=("c", "s"),
            dimension_semantics=(pltpu.PARALLEL,),
        )(i_hbm, o_hbm)

    return _gather_kernel(codebook, idx2d)


_CHUNKS = 4


def kernel(x, codebook):
    n = x.shape[0]
    cbt = codebook.T
    b2 = _codeword_norms(cbt)
    cbt_bf = cbt.astype(jnp.bfloat16)
    encs = [_assign(xc, cbt_bf, b2) for xc in jnp.split(x, _CHUNKS)]
    qs = [_sc_gather(codebook, enc) for enc in encs]
    encoded = jnp.concatenate(encs, axis=1).reshape(n)
    quantized = jnp.concatenate(qs)
    return (encoded, quantized)


# speedup vs baseline: 1.0152x; 1.0039x over previous
import functools

import jax
import jax.numpy as jnp
from jax.experimental import pallas as pl
from jax.experimental.pallas import tpu as pltpu
from jax.experimental.pallas import tpu_sc as plsc

_BN = 256

_PREC = jax.lax.Precision.DEFAULT


def _b2_body(cbt_ref, b2_ref):
    cbt = cbt_ref[...]
    b2_ref[...] = jnp.sum(cbt * cbt, axis=0, keepdims=True)


def _codeword_norms(cbt):
    d, k = cbt.shape
    return pl.pallas_call(
        _b2_body,
        out_shape=jax.ShapeDtypeStruct((1, k), jnp.float32),
    )(cbt)


def _assign_body(x_ref, cbt_ref, b2_ref, enc_ref):
    k = cbt_ref.shape[1]

    x = x_ref[...]
    a2 = jnp.sum(x * x, axis=1, keepdims=True)
    s = jax.lax.dot_general(
        x.astype(jnp.bfloat16), cbt_ref[...], (((1,), (0,)), ((), ())),
        preferred_element_type=jnp.float32, precision=_PREC,
    )
    d2 = (a2 - 2.0 * s) + b2_ref[...]
    m = jnp.min(d2, axis=1, keepdims=True)
    thr = jnp.where(m > 0, m * (1.0 + 0.9e-7), 0.0)
    iota = jax.lax.broadcasted_iota(jnp.int32, d2.shape, 1)
    idx = jnp.min(jnp.where(d2 <= thr, iota, k), axis=1, keepdims=True)
    enc_ref[...] = idx.reshape(1, idx.shape[0])


def _assign(x, cbt_bf, b2):
    n, d = x.shape
    k = cbt_bf.shape[1]
    return pl.pallas_call(
        _assign_body,
        grid=(n // _BN,),
        in_specs=[
            pl.BlockSpec((_BN, d), lambda i: (i, 0)),
            pl.BlockSpec((d, k), lambda i: (0, 0)),
            pl.BlockSpec((1, k), lambda i: (0, 0)),
        ],
        out_specs=pl.BlockSpec((1, _BN), lambda i: (0, i)),
        out_shape=jax.ShapeDtypeStruct((1, n), jnp.int32),
    )(x, cbt_bf, b2)


def _sc_gather(codebook, idx2d):
    n = idx2d.shape[1]
    d = codebook.shape[1]
    mesh = plsc.VectorSubcoreMesh(core_axis_name="c", subcore_axis_name="s")
    units = mesh.num_cores * mesh.num_subcores
    per = n // units

    tile = 64
    ntiles = per // tile

    @functools.partial(
        pl.kernel,
        out_type=jax.ShapeDtypeStruct((n, d), codebook.dtype),
        mesh=mesh,
        scratch_types=[
            pltpu.VMEM((per,), jnp.int32),
            pltpu.VMEM((2, tile, d), jnp.float32),
            pltpu.SemaphoreType.DMA,
            pltpu.SemaphoreType.DMA,
            pltpu.SemaphoreType.DMA,
        ],
    )
    def _gather_kernel(cb_hbm, i_hbm, o_hbm, iv, buf, isem, gsem, osems):
        c = jax.lax.axis_index("c")
        s = jax.lax.axis_index("s")
        base = (c * mesh.num_subcores + s) * per
        pltpu.async_copy(i_hbm.at[0, pl.ds(base, per)], iv, isem).wait()
        out_cps = []
        for t in range(ntiles):
            b = t % 2
            if t >= 2:
                out_cps[t - 2].wait()
            pltpu.async_copy(
                cb_hbm.at[iv.at[pl.ds(t * tile, tile)]], buf.at[b], gsem
            ).wait()
            out_cps.append(
                pltpu.async_copy(
                    buf.at[b], o_hbm.at[pl.ds(base + t * tile, tile)], osems
                )
            )
        out_cps[-2].wait()
        out_cps[-1].wait()

    return _gather_kernel(codebook, idx2d)


_CHUNKS = 1


def kernel(x, codebook):
    n = x.shape[0]
    cbt = codebook.T
    b2 = _codeword_norms(cbt)
    cbt_bf = cbt.astype(jnp.bfloat16)
    encs = [_assign(xc, cbt_bf, b2) for xc in jnp.split(x, _CHUNKS)]
    qs = [_sc_gather(codebook, enc) for enc in encs]
    encoded = jnp.concatenate(encs, axis=1).reshape(n)
    quantized = jnp.concatenate(qs)
    return (encoded, quantized)
